# Initial kernel scaffold; baseline (speedup 1.0000x reference)
import jax
import jax.numpy as jnp
from jax.experimental import pallas as pl


def kernel(x, edge_index, edge_weight, W_in, b_in, Ws0, bs0, Wn0, bn0, g0, be0, rm0, rv0, Ws1, bs1, Wn1, bn1, g1, be1, rm1, rv1, Ws2, bs2, Wn2, bn2, g2, be2, rm2, rv2):
    raise NotImplementedError("write your pallas kernel here")


# speedup vs baseline: 4.8390x; 4.8390x over previous
import functools

import jax
import jax.numpy as jnp
from jax import lax
from jax.experimental import pallas as pl
from jax.experimental.pallas import tpu as pltpu
from jax.experimental.pallas import tpu_sc as plsc

N = 10000
E = 320000
D = 128
NP_ = 10240
NC = 2
NS = 16
NW = NC * NS
C = 128
NCHUNKS = E // C
TPW = (NCHUNKS + NW - 1) // NW
RPT = NP_ // NS
ZR = 64


def _make_edge_kernel(with_deg: bool):
  out_type = [jax.ShapeDtypeStruct((NP_, D), jnp.float32),
              jax.ShapeDtypeStruct((NP_, D), jnp.float32)]
  if with_deg:
    out_type += [jax.ShapeDtypeStruct((NP_,), jnp.float32),
                 jax.ShapeDtypeStruct((NP_,), jnp.float32)]
  scratch = [
      pltpu.VMEM_SHARED((NP_, D), jnp.float32),
      pltpu.VMEM((C,), jnp.int32),
      pltpu.VMEM((C,), jnp.int32),
      pltpu.VMEM((C,), jnp.float32),
      pltpu.VMEM((C, D), jnp.float32),
      pltpu.VMEM((ZR, D), jnp.float32),
      pltpu.SemaphoreType.DMA,
  ]
  if with_deg:
    scratch.insert(1, pltpu.VMEM_SHARED((NP_,), jnp.float32))
    scratch.insert(2, pltpu.VMEM((ZR,), jnp.float32))

  mesh = plsc.VectorSubcoreMesh(core_axis_name="c", subcore_axis_name="s",
                                num_cores=NC, num_subcores=NS)

  def body(*refs):
    if with_deg:
      (h_hbm, row_hbm, col_hbm, ew_hbm, p0, p1, d0, d1,
       acc, dacc, dz, row_b, col_b, ew_b, gbuf, zbuf, sem) = refs
    else:
      (h_hbm, row_hbm, col_hbm, ew_hbm, p0, p1,
       acc, row_b, col_b, ew_b, gbuf, zbuf, sem) = refs
    cid = lax.axis_index("c")
    sid = lax.axis_index("s")
    wid = cid * NS + sid

    def zloop(r, _):
      z16 = jnp.zeros((16,), jnp.float32)
      for v in range(D // 16):
        zbuf[r, pl.ds(v * 16, 16)] = z16
      return 0
    lax.fori_loop(0, ZR, zloop, 0)
    for k in range(RPT // ZR):
      pltpu.sync_copy(zbuf, acc.at[pl.ds(sid * RPT + k * ZR, ZR)])
    if with_deg:
      z16 = jnp.zeros((16,), jnp.float32)
      for v in range(ZR // 16):
        dz[pl.ds(v * 16, 16)] = z16
      for k in range(RPT // ZR):
        pltpu.sync_copy(dz, dacc.at[pl.ds(sid * RPT + k * ZR, ZR)])
    plsc.subcore_barrier()

    def chunk_body(t, _):
      chunk = wid + NW * t

      @pl.when(chunk < NCHUNKS)
      def _():
        base = chunk * C
        pltpu.sync_copy(row_hbm.at[pl.ds(base, C)], row_b)
        pltpu.sync_copy(col_hbm.at[pl.ds(base, C)], col_b)
        pltpu.sync_copy(ew_hbm.at[pl.ds(base, C)], ew_b)
        pltpu.async_copy(h_hbm.at[row_b], gbuf, sem).wait()

        def mul_body(gidx, _):
          wv = ew_b[pl.ds(gidx * 16, 16)]
          for e16 in range(16):
            w = jnp.full((16,), wv[e16], jnp.float32)
            e = gidx * 16 + e16
            for v in range(D // 16):
              sl = pl.ds(v * 16, 16)
              gbuf[e, sl] = gbuf[e, sl] * w
          return 0
        lax.fori_loop(0, C // 16, mul_body, 0)
        pltpu.sync_copy(gbuf, acc.at[col_b], add=True)
        if with_deg:
          pltpu.sync_copy(ew_b, dacc.at[col_b], add=True)
      return 0

    lax.fori_loop(0, TPW, chunk_body, 0)
    plsc.subcore_barrier()

    sl = pl.ds(sid * RPT, RPT)

    @pl.when(cid == 0)
    def _():
      pltpu.sync_copy(acc.at[sl], p0.at[sl])
      if with_deg:
        pltpu.sync_copy(dacc.at[sl], d0.at[sl])

    @pl.when(cid == 1)
    def _():
      pltpu.sync_copy(acc.at[sl], p1.at[sl])
      if with_deg:
        pltpu.sync_copy(dacc.at[sl], d1.at[sl])

  return pl.kernel(body, out_type=out_type, mesh=mesh, scratch_types=scratch)


_edge_deg_kernel = _make_edge_kernel(with_deg=True)
_edge_kernel = _make_edge_kernel(with_deg=False)


def _input_body(x_ref, w_ref, b_ref, o_ref):
  y = lax.dot_general(x_ref[...], w_ref[...], (((1,), (1,)), ((), ())),
                      precision=lax.Precision.HIGHEST)
  o_ref[...] = jnp.maximum(y + b_ref[...][None, :], 0.0)


BROW = 1024
GRID = NP_ // BROW


def _tc_input(xp, W_in, b_in):
  return pl.pallas_call(
      _input_body,
      grid=(GRID,),
      in_specs=[
          pl.BlockSpec((BROW, D), lambda i: (i, 0)),
          pl.BlockSpec((D, D), lambda i: (0, 0)),
          pl.BlockSpec((D,), lambda i: (0,)),
      ],
      out_specs=pl.BlockSpec((BROW, D), lambda i: (i, 0)),
      out_shape=jax.ShapeDtypeStruct((NP_, D), jnp.float32),
  )(xp, W_in, b_in)


def _layer_body(do_relu, h_ref, p0_ref, p1_ref, d0_ref, d1_ref,
                ws_ref, bs_ref, wn_ref, bn_ref, g_ref, be_ref, rm_ref, rv_ref,
                o_ref):
  h = h_ref[...]
  agg = p0_ref[...] + p1_ref[...]
  deg = jnp.clip(d0_ref[...] + d1_ref[...], 1.0, None)
  xs = lax.dot_general(h, ws_ref[...], (((1,), (1,)), ((), ())),
                       precision=lax.Precision.HIGHEST) + bs_ref[...][None, :]
  xn = lax.dot_general(agg, wn_ref[...], (((1,), (1,)), ((), ())),
                       precision=lax.Precision.HIGHEST)
  xn = xn / deg[:, None] + bn_ref[...][None, :]
  y = xs + xn
  y = g_ref[...][None, :] * (y - rm_ref[...][None, :]) * lax.rsqrt(
      rv_ref[...][None, :] + 1e-5) + be_ref[...][None, :]
  if do_relu:
    y = jnp.maximum(y, 0.0)
  o_ref[...] = h + y


def _tc_layer(h, p0, p1, d0, d1, Ws, bs, Wn, bn, g, be, rm, rv, do_relu):
  vec = pl.BlockSpec((D,), lambda i: (0,))
  mat = pl.BlockSpec((D, D), lambda i: (0, 0))
  rows = pl.BlockSpec((BROW, D), lambda i: (i, 0))
  dvec = pl.BlockSpec((BROW,), lambda i: (i,))
  return pl.pallas_call(
      functools.partial(_layer_body, do_relu),
      grid=(GRID,),
      in_specs=[rows, rows, rows, dvec, dvec,
                mat, vec, mat, vec, vec, vec, vec, vec],
      out_specs=rows,
      out_shape=jax.ShapeDtypeStruct((NP_, D), jnp.float32),
  )(h, p0, p1, d0, d1, Ws, bs, Wn, bn, g, be, rm, rv)


def kernel(x, edge_index, edge_weight, W_in, b_in,
           Ws0, bs0, Wn0, bn0, g0, be0, rm0, rv0,
           Ws1, bs1, Wn1, bn1, g1, be1, rm1, rv1,
           Ws2, bs2, Wn2, bn2, g2, be2, rm2, rv2):
  row = edge_index[0]
  col = edge_index[1]
  xp = jnp.pad(x, ((0, NP_ - N), (0, 0)))
  h = _tc_input(xp, W_in, b_in)
  p0, p1, d0, d1 = _edge_deg_kernel(h, row, col, edge_weight)
  h = _tc_layer(h, p0, p1, d0, d1, Ws0, bs0, Wn0, bn0, g0, be0, rm0, rv0, True)
  p0, p1 = _edge_kernel(h, row, col, edge_weight)
  h = _tc_layer(h, p0, p1, d0, d1, Ws1, bs1, Wn1, bn1, g1, be1, rm1, rv1, True)
  p0, p1 = _edge_kernel(h, row, col, edge_weight)
  h = _tc_layer(h, p0, p1, d0, d1, Ws2, bs2, Wn2, bn2, g2, be2, rm2, rv2, False)
  return h[:N]


# speedup vs baseline: 10.6435x; 2.1995x over previous
import functools

import jax
import jax.numpy as jnp
from jax import lax
from jax.experimental import pallas as pl
from jax.experimental.pallas import tpu as pltpu
from jax.experimental.pallas import tpu_sc as plsc

N = 10000
E = 320000
D = 128
NP_ = 10240
NC = 2
NS = 16
NW = NC * NS
C = 128
CPT = 80
NCHP = NW * CPT
EP = NCHP * C
RPT = NP_ // NS
ZR = 64


def _make_edge_kernel(with_deg: bool):
  out_type = [jax.ShapeDtypeStruct((NP_, D), jnp.float32),
              jax.ShapeDtypeStruct((NP_, D), jnp.float32)]
  if with_deg:
    out_type += [jax.ShapeDtypeStruct((NP_,), jnp.float32),
                 jax.ShapeDtypeStruct((NP_,), jnp.float32)]
  names = ["acc", "ibuf", "ewbuf", "gbuf", "isem0", "isem1", "isem2",
           "isem3", "gsem0", "gsem1"]
  scratch = dict(
      acc=pltpu.VMEM_SHARED((NP_, D), jnp.float32),
      ibuf=pltpu.VMEM((4, 2, C), jnp.int32),
      ewbuf=pltpu.VMEM((4, C), jnp.float32),
      gbuf=pltpu.VMEM((2, C, D), jnp.float32),
      isem0=pltpu.SemaphoreType.DMA,
      isem1=pltpu.SemaphoreType.DMA,
      isem2=pltpu.SemaphoreType.DMA,
      isem3=pltpu.SemaphoreType.DMA,
      gsem0=pltpu.SemaphoreType.DMA,
      gsem1=pltpu.SemaphoreType.DMA,
  )
  if with_deg:
    scratch["dacc"] = pltpu.VMEM_SHARED((NP_,), jnp.float32)
    scratch["dz"] = pltpu.VMEM((RPT,), jnp.float32)
    names += ["dacc", "dz"]

  mesh = plsc.VectorSubcoreMesh(core_axis_name="c", subcore_axis_name="s",
                                num_cores=NC, num_subcores=NS)

  def body(h_hbm, pk_hbm, ew_hbm, *refs):
    nout = 4 if with_deg else 2
    outs = refs[:nout]
    sc = dict(zip(names, refs[nout:]))
    acc, ibuf, gbuf = sc["acc"], sc["ibuf"], sc["gbuf"]
    ewbuf = sc["ewbuf"]
    isems = (sc["isem0"], sc["isem1"], sc["isem2"], sc["isem3"])
    gsems = (sc["gsem0"], sc["gsem1"])
    cid = lax.axis_index("c")
    sid = lax.axis_index("s")
    wid = cid * NS + sid
    start = wid * CPT

    def zloop(r, _):
      z16 = jnp.zeros((16,), jnp.float32)
      for v in range(D // 16):
        gbuf[0, r, pl.ds(v * 16, 16)] = z16
      return 0
    lax.fori_loop(0, C, zloop, 0)
    for k in range(RPT // C):
      pltpu.sync_copy(gbuf.at[0], acc.at[pl.ds(sid * RPT + k * C, C)])
    if with_deg:
      dacc, dz = sc["dacc"], sc["dz"]
      z16 = jnp.zeros((16,), jnp.float32)
      for v in range(RPT // 16):
        dz[pl.ds(v * 16, 16)] = z16
      pltpu.sync_copy(dz, dacc.at[pl.ds(sid * RPT, RPT)])

    def load_idx(t, r):
      pltpu.async_copy(pk_hbm.at[start + t], ibuf.at[r], isems[r])
      pltpu.async_copy(ew_hbm.at[start + t], ewbuf.at[r], isems[r])

    def wait_idx(t, r):
      pltpu.make_async_copy(pk_hbm.at[start + t], ibuf.at[r],
                            isems[r]).wait()
      pltpu.make_async_copy(ew_hbm.at[start + t], ewbuf.at[r],
                            isems[r]).wait()

    def gather(t, r, b):
      pltpu.async_copy(h_hbm.at[ibuf.at[r, 0]], gbuf.at[b], gsems[b])

    def wait_gather(t, r, b):
      pltpu.make_async_copy(h_hbm.at[ibuf.at[r, 0]], gbuf.at[b],
                            gsems[b]).wait()

    def mul(r, b):
      def mul_body(gidx, _):
        wv = ewbuf[r, pl.ds(gidx * 16, 16)]
        for e16 in range(16):
          w = jnp.full((16,), wv[e16], jnp.float32)
          e = gidx * 16 + e16
          for v in range(D // 16):
            sl = pl.ds(v * 16, 16)
            gbuf[b, e, sl] = gbuf[b, e, sl] * w
        return 0
      lax.fori_loop(0, C // 16, mul_body, 0)

    plsc.subcore_barrier()

    for t in range(4):
      load_idx(t, t)
    wait_idx(0, 0)
    gather(0, 0, 0)

    def step(s, _):
      for b4 in range(4):
        t = 4 * s + b4
        r = b4
        rn = (b4 + 1) % 4
        b = b4 % 2
        bn = 1 - b

        @pl.when(t + 1 < CPT)
        def _():
          wait_idx(t + 1, rn)
          gather(t + 1, rn, bn)
        wait_gather(t, r, b)
        mul(r, b)
        pltpu.sync_copy(gbuf.at[b], acc.at[ibuf.at[r, 1]], add=True)
        if with_deg:
          pltpu.sync_copy(ewbuf.at[r], dacc.at[ibuf.at[r, 1]], add=True)

        @pl.when(t + 4 < CPT)
        def _():
          load_idx(t + 4, r)
      return 0

    lax.fori_loop(0, CPT // 4, step, 0)
    plsc.subcore_barrier()

    sl = pl.ds(sid * RPT, RPT)

    @pl.when(cid == 0)
    def _():
      pltpu.sync_copy(acc.at[sl], outs[0].at[sl])
      if with_deg:
        pltpu.sync_copy(dacc.at[sl], outs[2].at[sl])

    @pl.when(cid == 1)
    def _():
      pltpu.sync_copy(acc.at[sl], outs[1].at[sl])
      if with_deg:
        pltpu.sync_copy(dacc.at[sl], outs[3].at[sl])

  return pl.kernel(body, out_type=out_type, mesh=mesh,
                   scratch_types=list(scratch.values()))


_edge_deg_kernel = _make_edge_kernel(with_deg=True)
_edge_kernel = _make_edge_kernel(with_deg=False)


def _input_body(x_ref, w_ref, b_ref, o_ref):
  y = lax.dot_general(x_ref[...], w_ref[...], (((1,), (1,)), ((), ())),
                      precision=lax.Precision.HIGHEST)
  o_ref[...] = jnp.maximum(y + b_ref[...][None, :], 0.0)


BROW = 1024
GRID = NP_ // BROW


def _tc_input(xp, W_in, b_in):
  return pl.pallas_call(
      _input_body,
      grid=(GRID,),
      in_specs=[
          pl.BlockSpec((BROW, D), lambda i: (i, 0)),
          pl.BlockSpec((D, D), lambda i: (0, 0)),
          pl.BlockSpec((D,), lambda i: (0,)),
      ],
      out_specs=pl.BlockSpec((BROW, D), lambda i: (i, 0)),
      out_shape=jax.ShapeDtypeStruct((NP_, D), jnp.float32),
  )(xp, W_in, b_in)


def _layer_body(do_relu, h_ref, p0_ref, p1_ref, d0_ref, d1_ref,
                ws_ref, bs_ref, wn_ref, bn_ref, g_ref, be_ref, rm_ref, rv_ref,
                o_ref):
  h = h_ref[...]
  agg = p0_ref[...] + p1_ref[...]
  deg = jnp.clip(d0_ref[...] + d1_ref[...], 1.0, None)
  xs = lax.dot_general(h, ws_ref[...], (((1,), (1,)), ((), ())),
                       precision=lax.Precision.HIGHEST) + bs_ref[...][None, :]
  xn = lax.dot_general(agg, wn_ref[...], (((1,), (1,)), ((), ())),
                       precision=lax.Precision.HIGHEST)
  xn = xn / deg[:, None] + bn_ref[...][None, :]
  y = xs + xn
  y = g_ref[...][None, :] * (y - rm_ref[...][None, :]) * lax.rsqrt(
      rv_ref[...][None, :] + 1e-5) + be_ref[...][None, :]
  if do_relu:
    y = jnp.maximum(y, 0.0)
  o_ref[...] = h + y


def _tc_layer(h, p0, p1, d0, d1, Ws, bs, Wn, bn, g, be, rm, rv, do_relu):
  vec = pl.BlockSpec((D,), lambda i: (0,))
  mat = pl.BlockSpec((D, D), lambda i: (0, 0))
  rows = pl.BlockSpec((BROW, D), lambda i: (i, 0))
  dvec = pl.BlockSpec((BROW,), lambda i: (i,))
  return pl.pallas_call(
      functools.partial(_layer_body, do_relu),
      grid=(GRID,),
      in_specs=[rows, rows, rows, dvec, dvec,
                mat, vec, mat, vec, vec, vec, vec, vec],
      out_specs=rows,
      out_shape=jax.ShapeDtypeStruct((NP_, D), jnp.float32),
  )(h, p0, p1, d0, d1, Ws, bs, Wn, bn, g, be, rm, rv)


def kernel(x, edge_index, edge_weight, W_in, b_in,
           Ws0, bs0, Wn0, bn0, g0, be0, rm0, rv0,
           Ws1, bs1, Wn1, bn1, g1, be1, rm1, rv1,
           Ws2, bs2, Wn2, bn2, g2, be2, rm2, rv2):
  pad = EP - E
  fill = jnp.arange(pad, dtype=jnp.int32) % N
  row = jnp.concatenate([edge_index[0], fill]).reshape(NCHP, C)
  col = jnp.concatenate([edge_index[1], fill]).reshape(NCHP, C)
  ew = jnp.concatenate(
      [edge_weight, jnp.zeros((pad,), jnp.float32)]).reshape(NCHP, C)
  pk = jnp.stack([row, col], axis=1)
  xp = jnp.pad(x, ((0, NP_ - N), (0, 0)))
  h = _tc_input(xp, W_in, b_in)
  p0, p1, d0, d1 = _edge_deg_kernel(h, pk, ew)
  h = _tc_layer(h, p0, p1, d0, d1, Ws0, bs0, Wn0, bn0, g0, be0, rm0, rv0, True)
  p0, p1 = _edge_kernel(h, pk, ew)
  h = _tc_layer(h, p0, p1, d0, d1, Ws1, bs1, Wn1, bn1, g1, be1, rm1, rv1, True)
  p0, p1 = _edge_kernel(h, pk, ew)
  h = _tc_layer(h, p0, p1, d0, d1, Ws2, bs2, Wn2, bn2, g2, be2, rm2, rv2, False)
  return h[:N]


# speedup vs baseline: 10.7154x; 1.0067x over previous
import functools

import jax
import jax.numpy as jnp
from jax import lax
from jax.experimental import pallas as pl
from jax.experimental.pallas import tpu as pltpu
from jax.experimental.pallas import tpu_sc as plsc

N = 10000
E = 320000
D = 128
NP_ = 10240
NC = 2
NS = 16
NW = NC * NS
C = 128
CPT = 80
NCHP = NW * CPT
EP = NCHP * C
RPT = NP_ // NS
ZR = 64


def _make_edge_kernel(with_deg: bool):
  out_type = [jax.ShapeDtypeStruct((NP_, D), jnp.float32),
              jax.ShapeDtypeStruct((NP_, D), jnp.float32)]
  if with_deg:
    out_type += [jax.ShapeDtypeStruct((NP_,), jnp.float32),
                 jax.ShapeDtypeStruct((NP_,), jnp.float32)]
  names = ["acc", "ibuf", "ewbuf", "gbuf", "isem0", "isem1", "isem2",
           "isem3", "gsem0", "gsem1", "ssem0", "ssem1"]
  scratch = dict(
      acc=pltpu.VMEM_SHARED((NP_, D), jnp.float32),
      ibuf=pltpu.VMEM((4, 2, C), jnp.int32),
      ewbuf=pltpu.VMEM((4, C), jnp.float32),
      gbuf=pltpu.VMEM((2, C, D), jnp.float32),
      isem0=pltpu.SemaphoreType.DMA,
      isem1=pltpu.SemaphoreType.DMA,
      isem2=pltpu.SemaphoreType.DMA,
      isem3=pltpu.SemaphoreType.DMA,
      gsem0=pltpu.SemaphoreType.DMA,
      gsem1=pltpu.SemaphoreType.DMA,
      ssem0=pltpu.SemaphoreType.DMA,
      ssem1=pltpu.SemaphoreType.DMA,
  )
  if with_deg:
    scratch["dacc"] = pltpu.VMEM_SHARED((NP_,), jnp.float32)
    scratch["dz"] = pltpu.VMEM((RPT,), jnp.float32)
    names += ["dacc", "dz"]

  mesh = plsc.VectorSubcoreMesh(core_axis_name="c", subcore_axis_name="s",
                                num_cores=NC, num_subcores=NS)

  def body(h_hbm, pk_hbm, ew_hbm, *refs):
    nout = 4 if with_deg else 2
    outs = refs[:nout]
    sc = dict(zip(names, refs[nout:]))
    acc, ibuf, gbuf = sc["acc"], sc["ibuf"], sc["gbuf"]
    ewbuf = sc["ewbuf"]
    isems = (sc["isem0"], sc["isem1"], sc["isem2"], sc["isem3"])
    gsems = (sc["gsem0"], sc["gsem1"])
    ssems = (sc["ssem0"], sc["ssem1"])
    cid = lax.axis_index("c")
    sid = lax.axis_index("s")
    wid = cid * NS + sid
    start = wid * CPT

    def zloop(r, _):
      z16 = jnp.zeros((16,), jnp.float32)
      for v in range(D // 16):
        gbuf[0, r, pl.ds(v * 16, 16)] = z16
      return 0
    lax.fori_loop(0, C, zloop, 0)
    for k in range(RPT // C):
      pltpu.sync_copy(gbuf.at[0], acc.at[pl.ds(sid * RPT + k * C, C)])
    if with_deg:
      dacc, dz = sc["dacc"], sc["dz"]
      z16 = jnp.zeros((16,), jnp.float32)
      for v in range(RPT // 16):
        dz[pl.ds(v * 16, 16)] = z16
      pltpu.sync_copy(dz, dacc.at[pl.ds(sid * RPT, RPT)])

    def load_idx(t, r):
      pltpu.async_copy(pk_hbm.at[start + t], ibuf.at[r], isems[r])
      pltpu.async_copy(ew_hbm.at[start + t], ewbuf.at[r], isems[r])

    def wait_idx(t, r):
      pltpu.make_async_copy(pk_hbm.at[start + t], ibuf.at[r],
                            isems[r]).wait()
      pltpu.make_async_copy(ew_hbm.at[start + t], ewbuf.at[r],
                            isems[r]).wait()

    def gather(t, r, b):
      pltpu.async_copy(h_hbm.at[ibuf.at[r, 0]], gbuf.at[b], gsems[b])

    def wait_gather(t, r, b):
      pltpu.make_async_copy(h_hbm.at[ibuf.at[r, 0]], gbuf.at[b],
                            gsems[b]).wait()

    def scatter(r, b):
      pltpu.async_copy(gbuf.at[b], acc.at[ibuf.at[r, 1]], ssems[b], add=True)
      if with_deg:
        pltpu.async_copy(ewbuf.at[r], sc["dacc"].at[ibuf.at[r, 1]], ssems[b],
                         add=True)

    def wait_scatter(r, b):
      pltpu.make_async_copy(gbuf.at[b], acc.at[ibuf.at[r, 1]],
                            ssems[b]).wait()
      if with_deg:
        pltpu.make_async_copy(ewbuf.at[r], sc["dacc"].at[ibuf.at[r, 1]],
                              ssems[b]).wait()

    def mul(r, b):
      def mul_body(gidx, _):
        wv = ewbuf[r, pl.ds(gidx * 16, 16)]
        for e16 in range(16):
          w = jnp.full((16,), wv[e16], jnp.float32)
          e = gidx * 16 + e16
          for v in range(D // 16):
            sl = pl.ds(v * 16, 16)
            gbuf[b, e, sl] = gbuf[b, e, sl] * w
        return 0
      lax.fori_loop(0, C // 16, mul_body, 0)

    plsc.subcore_barrier()

    for t in range(4):
      load_idx(t, t)
    wait_idx(0, 0)
    gather(0, 0, 0)

    def step(s, _):
      for b4 in range(4):
        t = 4 * s + b4
        r = b4
        rn = (b4 + 1) % 4
        b = b4 % 2
        bn = 1 - b

        rp = (b4 - 1) % 4

        @pl.when(t + 1 < CPT)
        def _():
          wait_idx(t + 1, rn)

          @pl.when(t >= 1)
          def _():
            wait_scatter(rp, bn)

            @pl.when(t + 3 < CPT)
            def _():
              load_idx(t + 3, rp)
          gather(t + 1, rn, bn)
        wait_gather(t, r, b)
        mul(r, b)
        scatter(r, b)
      return 0

    lax.fori_loop(0, CPT // 4, step, 0)
    wait_scatter(2, 0)
    wait_scatter(3, 1)
    plsc.subcore_barrier()

    sl = pl.ds(sid * RPT, RPT)

    @pl.when(cid == 0)
    def _():
      pltpu.sync_copy(acc.at[sl], outs[0].at[sl])
      if with_deg:
        pltpu.sync_copy(dacc.at[sl], outs[2].at[sl])

    @pl.when(cid == 1)
    def _():
      pltpu.sync_copy(acc.at[sl], outs[1].at[sl])
      if with_deg:
        pltpu.sync_copy(dacc.at[sl], outs[3].at[sl])

  return pl.kernel(body, out_type=out_type, mesh=mesh,
                   scratch_types=list(scratch.values()))


_edge_deg_kernel = _make_edge_kernel(with_deg=True)
_edge_kernel = _make_edge_kernel(with_deg=False)


def _input_body(x_ref, w_ref, b_ref, o_ref):
  y = lax.dot_general(x_ref[...], w_ref[...], (((1,), (1,)), ((), ())),
                      precision=lax.Precision.HIGHEST)
  o_ref[...] = jnp.maximum(y + b_ref[...][None, :], 0.0)


BROW = 1024
GRID = NP_ // BROW


def _tc_input(xp, W_in, b_in):
  return pl.pallas_call(
      _input_body,
      grid=(GRID,),
      in_specs=[
          pl.BlockSpec((BROW, D), lambda i: (i, 0)),
          pl.BlockSpec((D, D), lambda i: (0, 0)),
          pl.BlockSpec((D,), lambda i: (0,)),
      ],
      out_specs=pl.BlockSpec((BROW, D), lambda i: (i, 0)),
      out_shape=jax.ShapeDtypeStruct((NP_, D), jnp.float32),
  )(xp, W_in, b_in)


def _layer_body(do_relu, h_ref, p0_ref, p1_ref, d0_ref, d1_ref,
                ws_ref, bs_ref, wn_ref, bn_ref, g_ref, be_ref, rm_ref, rv_ref,
                o_ref):
  h = h_ref[...]
  agg = p0_ref[...] + p1_ref[...]
  deg = jnp.clip(d0_ref[...] + d1_ref[...], 1.0, None)
  xs = lax.dot_general(h, ws_ref[...], (((1,), (1,)), ((), ())),
                       precision=lax.Precision.HIGHEST) + bs_ref[...][None, :]
  xn = lax.dot_general(agg, wn_ref[...], (((1,), (1,)), ((), ())),
                       precision=lax.Precision.HIGHEST)
  xn = xn / deg[:, None] + bn_ref[...][None, :]
  y = xs + xn
  y = g_ref[...][None, :] * (y - rm_ref[...][None, :]) * lax.rsqrt(
      rv_ref[...][None, :] + 1e-5) + be_ref[...][None, :]
  if do_relu:
    y = jnp.maximum(y, 0.0)
  o_ref[...] = h + y


def _tc_layer(h, p0, p1, d0, d1, Ws, bs, Wn, bn, g, be, rm, rv, do_relu):
  vec = pl.BlockSpec((D,), lambda i: (0,))
  mat = pl.BlockSpec((D, D), lambda i: (0, 0))
  rows = pl.BlockSpec((BROW, D), lambda i: (i, 0))
  dvec = pl.BlockSpec((BROW,), lambda i: (i,))
  return pl.pallas_call(
      functools.partial(_layer_body, do_relu),
      grid=(GRID,),
      in_specs=[rows, rows, rows, dvec, dvec,
                mat, vec, mat, vec, vec, vec, vec, vec],
      out_specs=rows,
      out_shape=jax.ShapeDtypeStruct((NP_, D), jnp.float32),
  )(h, p0, p1, d0, d1, Ws, bs, Wn, bn, g, be, rm, rv)


def kernel(x, edge_index, edge_weight, W_in, b_in,
           Ws0, bs0, Wn0, bn0, g0, be0, rm0, rv0,
           Ws1, bs1, Wn1, bn1, g1, be1, rm1, rv1,
           Ws2, bs2, Wn2, bn2, g2, be2, rm2, rv2):
  pad = EP - E
  fill = jnp.arange(pad, dtype=jnp.int32) % N
  row = jnp.concatenate([edge_index[0], fill]).reshape(NCHP, C)
  col = jnp.concatenate([edge_index[1], fill]).reshape(NCHP, C)
  ew = jnp.concatenate(
      [edge_weight, jnp.zeros((pad,), jnp.float32)]).reshape(NCHP, C)
  pk = jnp.stack([row, col], axis=1)
  xp = jnp.pad(x, ((0, NP_ - N), (0, 0)))
  h = _tc_input(xp, W_in, b_in)
  p0, p1, d0, d1 = _edge_deg_kernel(h, pk, ew)
  h = _tc_layer(h, p0, p1, d0, d1, Ws0, bs0, Wn0, bn0, g0, be0, rm0, rv0, True)
  p0, p1 = _edge_kernel(h, pk, ew)
  h = _tc_layer(h, p0, p1, d0, d1, Ws1, bs1, Wn1, bn1, g1, be1, rm1, rv1, True)
  p0, p1 = _edge_kernel(h, pk, ew)
  h = _tc_layer(h, p0, p1, d0, d1, Ws2, bs2, Wn2, bn2, g2, be2, rm2, rv2, False)
  return h[:N]
